# baseline (speedup 1.0000x reference)
import functools

import jax
import jax.numpy as jnp
from jax import lax
from jax.experimental import pallas as pl
from jax.experimental.pallas import tpu as pltpu
from jax.experimental.pallas import tpu_sc as plsc

_L = 16
_NB = 1024
_CH = 8000
_G = 4

_NEG_INF_PAIR = -8323200


def _pack_rows(top, bot):
    t = lax.bitcast_convert_type(top.astype(jnp.bfloat16), jnp.uint16)
    b = lax.bitcast_convert_type(bot.astype(jnp.bfloat16), jnp.uint16)
    u = t.astype(jnp.uint32) | (b.astype(jnp.uint32) << 16)
    return lax.bitcast_convert_type(u, jnp.int32)


def _unpack_rows(p):
    u = lax.bitcast_convert_type(p, jnp.uint32)
    lo = lax.bitcast_convert_type((u & 0xFFFF).astype(jnp.uint16),
                                  jnp.bfloat16).astype(jnp.float32)
    hi = lax.bitcast_convert_type((u >> 16).astype(jnp.uint16),
                                  jnp.bfloat16).astype(jnp.float32)
    return jnp.concatenate([lo, hi], axis=0)


def _tc1_body(x_ref, u_ref, v_ref, b_ref, xT_ref, a_ref, bp_ref):
    xT = x_ref[...].T
    xT_ref[...] = xT
    C = xT.shape[0]
    a_ref[...] = jnp.dot(u_ref[...], xT, preferred_element_type=jnp.float32) + b_ref[...]
    bm = jnp.dot(v_ref[...], xT, preferred_element_type=jnp.float32)
    bp_ref[...] = _pack_rows(bm[:C // 2], bm[C // 2:])


def _lrelu_gate(s, a):
    z = a + s
    h = jnp.where(z >= 0, z, 0.2 * z)
    return jnp.where(s == -jnp.inf, 0.0, h)


def _tc2_body(xT_ref, sp_ref, a0_ref, u1x_ref, u1h_ref, v1x_ref, v1h_ref,
              b_ref, h0_ref, a1_ref, b1p_ref):
    s = _unpack_rows(sp_ref[...])
    h0 = _lrelu_gate(s, a0_ref[...])
    h0_ref[...] = h0
    xT = xT_ref[...]
    C = xT.shape[0]
    dot = lambda w, m: jnp.dot(w, m, preferred_element_type=jnp.float32)
    a1_ref[...] = dot(u1x_ref[...], xT) + dot(u1h_ref[...], h0) + b_ref[...]
    b1 = dot(v1x_ref[...], xT) + dot(v1h_ref[...], h0)
    b1p_ref[...] = _pack_rows(b1[:C // 2], b1[C // 2:])


def _tc3_body(xT_ref, h0_ref, sp_ref, a1_ref, wx_ref, wh0_ref, wh1_ref,
              b_ref, out_ref):
    s = _unpack_rows(sp_ref[...])
    h1 = _lrelu_gate(s, a1_ref[...])
    xT = xT_ref[...]
    dot = lambda w, m: jnp.dot(w, m, preferred_element_type=jnp.float32)
    resT = (dot(wx_ref[...], xT) + dot(wh0_ref[...], h0_ref[...])
            + dot(wh1_ref[...], h1) + b_ref[...] + xT)
    out_ref[...] = resT.T


def _make_tc_calls(N, C, interpret=False):
    g = N // _NB
    full = pl.BlockSpec((C, C), lambda i: (0, 0))
    bias = pl.BlockSpec((C, 1), lambda i: (0, 0))
    colT = pl.BlockSpec((C, _NB), lambda i: (0, i))
    colP = pl.BlockSpec((C // 2, _NB), lambda i: (0, i))
    rows = pl.BlockSpec((_NB, C), lambda i: (i, 0))
    fTN = jax.ShapeDtypeStruct((C, N), jnp.float32)
    iPN = jax.ShapeDtypeStruct((C // 2, N), jnp.int32)

    tc1 = pl.pallas_call(
        _tc1_body, grid=(g,),
        in_specs=[rows, full, full, bias],
        out_specs=[colT, colT, colP],
        out_shape=[fTN, fTN, iPN],
        interpret=interpret)
    tc2 = pl.pallas_call(
        _tc2_body, grid=(g,),
        in_specs=[colT, colP, colT, full, full, full, full, bias],
        out_specs=[colT, colT, colP],
        out_shape=[fTN, fTN, iPN],
        interpret=interpret)
    tc3 = pl.pallas_call(
        _tc3_body, grid=(g,),
        in_specs=[colT, colT, colP, colT, full, full, full, bias],
        out_specs=rows,
        out_shape=jax.ShapeDtypeStruct((N, C), jnp.float32),
        interpret=interpret)
    return tc1, tc2, tc3


def _make_segmax(N, C, E):
    info = plsc.get_sparse_core_info()
    NC, NS = info.num_cores, info.num_subcores
    NW = NC * NS
    P = C // 2
    assert P % NW == 0
    RPW = P // NW
    assert N % _L == 0 and (RPW * N) % 8 == 0
    assert E % _CH == 0 and _CH % (_G * _L) == 0
    mesh = plsc.VectorSubcoreMesh(core_axis_name="c", subcore_axis_name="s")

    @functools.partial(
        pl.kernel, mesh=mesh,
        out_type=jax.ShapeDtypeStruct((P * N,), jnp.int32),
        compiler_params=pltpu.CompilerParams(needs_layout_passes=False),
        scratch_types=(
            [pltpu.VMEM((N,), jnp.int32)] * RPW
            + [pltpu.VMEM((N,), jnp.int32)] * RPW
            + [
                pltpu.VMEM((_CH,), jnp.int32),
                pltpu.VMEM((_CH,), jnp.int32),
            ]))
    def segmax(bp_hbm, src_hbm, dst_hbm, out_hbm, *scratch):
        b_v = scratch[:RPW]
        s_v = scratch[RPW:2 * RPW]
        src_v, dst_v = scratch[2 * RPW], scratch[2 * RPW + 1]
        wid = lax.axis_index("s") * NC + lax.axis_index("c")
        fbase = wid * RPW * N
        for j in range(RPW):
            pltpu.sync_copy(bp_hbm.at[pl.ds(fbase + j * N, N)], b_v[j])

        ninf = jnp.full((_L,), _NEG_INF_PAIR, jnp.int32)

        def init_body(i, c):
            for j in range(RPW):
                s_v[j][pl.ds(i * _L, _L)] = ninf
            return c
        lax.fori_loop(0, N // _L, init_body, 0)

        def pairmax(cur_i, val_i):
            m = jnp.maximum(plsc.bitcast(cur_i, jnp.bfloat16),
                            plsc.bitcast(val_i, jnp.bfloat16))
            mi = plsc.bitcast(m, jnp.int32)
            return mi, mi != cur_i

        def onepass(off):
            s_idx = src_v[pl.ds(off, _L)]
            d_idx = dst_v[pl.ds(off, _L)]
            vals = [plsc.load_gather(b_v[j], [s_idx]) for j in range(RPW)]
            cnt, last = plsc.scan_count(d_idx)
            dup = jnp.logical_or(jnp.logical_not(last), cnt > 0)
            curs = [plsc.load_gather(s_v[j], [d_idx]) for j in range(RPW)]
            chg = None
            for j in range(RPW):
                mi, ch = pairmax(curs[j], vals[j])
                plsc.store_scatter(s_v[j], [d_idx], mi, mask=ch)
                chg = ch if chg is None else jnp.logical_or(chg, ch)
            return jnp.logical_and(dup, chg)

        def fixpoint(off):
            s_idx = src_v[pl.ds(off, _L)]
            d_idx = dst_v[pl.ds(off, _L)]
            vals = [plsc.load_gather(b_v[j], [s_idx]) for j in range(RPW)]
            cnt, _ = plsc.scan_count(d_idx)
            hasdup = jnp.any(cnt > 0)

            def fix_body(_):
                chg = None
                for j in range(RPW):
                    cur = plsc.load_gather(s_v[j], [d_idx])
                    mi, ch = pairmax(cur, vals[j])
                    plsc.store_scatter(s_v[j], [d_idx], mi, mask=ch)
                    chg = ch if chg is None else jnp.logical_or(chg, ch)
                return jnp.logical_and(hasdup, jnp.any(chg))
            lax.while_loop(lambda r: r, fix_body, jnp.bool_(True))

        def group_body(gi, c):
            base = gi * (_G * _L)
            conflict = onepass(base)
            for u in range(1, _G):
                conflict = jnp.logical_or(conflict, onepass(base + u * _L))

            @pl.when(jnp.any(conflict))
            def _():
                for u in range(_G):
                    fixpoint(base + u * _L)
            return c

        def chunk_body(ci, c):
            pltpu.sync_copy(src_hbm.at[pl.ds(ci * _CH, _CH)], src_v)
            pltpu.sync_copy(dst_hbm.at[pl.ds(ci * _CH, _CH)], dst_v)
            ng = _CH // (_G * _L)
            lax.fori_loop(0, 2 * ng, lambda gi, c: group_body(gi % ng, c), 0)
            return c
        lax.fori_loop(0, E // _CH, chunk_body, 0)

        for j in range(RPW):
            pltpu.sync_copy(s_v[j], out_hbm.at[pl.ds(fbase + j * N, N)])

    return segmax


def kernel(x, edge_index, W0, b0, W1, b1, Wd, bd):
    Nin, C = x.shape
    E = edge_index.shape[1]
    src = edge_index[0]
    dst = edge_index[1]
    N = ((Nin + _NB - 1) // _NB) * _NB
    if N != Nin:
        x = jnp.pad(x, ((0, N - Nin), (0, 0)))
    if E % _CH:
        pad = _CH - E % _CH
        src = jnp.concatenate([src, jnp.broadcast_to(src[-1:], (pad,))])
        dst = jnp.concatenate([dst, jnp.broadcast_to(dst[-1:], (pad,))])
        E += pad

    U0t = (W0[:C] - W0[C:]).T
    V0t = W0[C:].T
    U1 = W1[:2 * C] - W1[2 * C:]
    V1 = W1[2 * C:]
    U1xt, U1ht = U1[:C].T, U1[C:].T
    V1xt, V1ht = V1[:C].T, V1[C:].T
    Wxt, Wh0t, Wh1t = Wd[:C].T, Wd[C:2 * C].T, Wd[2 * C:].T
    b0c = b0.reshape(C, 1)
    b1c = b1.reshape(C, 1)
    bdc = bd.reshape(C, 1)

    tc1, tc2, tc3 = _make_tc_calls(N, C)
    segmax = _make_segmax(N, C, E)

    xT, A0T, B0P = tc1(x, U0t, V0t, b0c)
    S0P = segmax(B0P.reshape(-1), src, dst).reshape(C // 2, N)
    h0T, A1T, B1P = tc2(xT, S0P, A0T, U1xt, U1ht, V1xt, V1ht, b1c)
    S1P = segmax(B1P.reshape(-1), src, dst).reshape(C // 2, N)
    out = tc3(xT, h0T, S1P, A1T, Wxt, Wh0t, Wh1t, bdc)
    return out[:Nin]


# speedup vs baseline: 1.2754x; 1.2754x over previous
import functools

import jax
import jax.numpy as jnp
from jax import lax
from jax.experimental import pallas as pl
from jax.experimental.pallas import tpu as pltpu
from jax.experimental.pallas import tpu_sc as plsc

_L = 16
_NB = 1024
_CH = 8192
_G = 4

_NEG_INF_PAIR = -8323200


def _pack_rows(top, bot):
    t = lax.bitcast_convert_type(top.astype(jnp.bfloat16), jnp.uint16)
    b = lax.bitcast_convert_type(bot.astype(jnp.bfloat16), jnp.uint16)
    u = t.astype(jnp.uint32) | (b.astype(jnp.uint32) << 16)
    return lax.bitcast_convert_type(u, jnp.int32)


def _unpack_rows(p):
    u = lax.bitcast_convert_type(p, jnp.uint32)
    lo = lax.bitcast_convert_type((u & 0xFFFF).astype(jnp.uint16),
                                  jnp.bfloat16).astype(jnp.float32)
    hi = lax.bitcast_convert_type((u >> 16).astype(jnp.uint16),
                                  jnp.bfloat16).astype(jnp.float32)
    return jnp.concatenate([lo, hi], axis=0)


def _pack_edges_body(src_ref, dst_ref, out_ref):
    out_ref[...] = src_ref[...] | (dst_ref[...] << 16)


def _tc1_body(x_ref, u_ref, v_ref, b_ref, xT_ref, a_ref, bp_ref):
    xT = x_ref[...].T
    xT_ref[...] = xT
    C = xT.shape[0]
    a_ref[...] = jnp.dot(u_ref[...], xT, preferred_element_type=jnp.float32) + b_ref[...]
    bm = jnp.dot(v_ref[...], xT, preferred_element_type=jnp.float32)
    bp_ref[...] = _pack_rows(bm[:C // 2], bm[C // 2:])


def _lrelu_gate(s, a):
    z = a + s
    h = jnp.where(z >= 0, z, 0.2 * z)
    return jnp.where(s == -jnp.inf, 0.0, h)


def _tc2_body(xT_ref, sp_ref, a0_ref, u1x_ref, u1h_ref, v1x_ref, v1h_ref,
              b_ref, h0_ref, a1_ref, b1p_ref):
    s = _unpack_rows(sp_ref[...])
    h0 = _lrelu_gate(s, a0_ref[...])
    h0_ref[...] = h0
    xT = xT_ref[...]
    C = xT.shape[0]
    dot = lambda w, m: jnp.dot(w, m, preferred_element_type=jnp.float32)
    a1_ref[...] = dot(u1x_ref[...], xT) + dot(u1h_ref[...], h0) + b_ref[...]
    b1 = dot(v1x_ref[...], xT) + dot(v1h_ref[...], h0)
    b1p_ref[...] = _pack_rows(b1[:C // 2], b1[C // 2:])


def _tc3_body(xT_ref, h0_ref, sp_ref, a1_ref, wx_ref, wh0_ref, wh1_ref,
              b_ref, out_ref):
    s = _unpack_rows(sp_ref[...])
    h1 = _lrelu_gate(s, a1_ref[...])
    xT = xT_ref[...]
    dot = lambda w, m: jnp.dot(w, m, preferred_element_type=jnp.float32)
    resT = (dot(wx_ref[...], xT) + dot(wh0_ref[...], h0_ref[...])
            + dot(wh1_ref[...], h1) + b_ref[...] + xT)
    out_ref[...] = resT.T


def _make_tc_calls(N, C, interpret=False):
    g = N // _NB
    full = pl.BlockSpec((C, C), lambda i: (0, 0))
    bias = pl.BlockSpec((C, 1), lambda i: (0, 0))
    colT = pl.BlockSpec((C, _NB), lambda i: (0, i))
    colP = pl.BlockSpec((C // 2, _NB), lambda i: (0, i))
    rows = pl.BlockSpec((_NB, C), lambda i: (i, 0))
    fTN = jax.ShapeDtypeStruct((C, N), jnp.float32)
    iPN = jax.ShapeDtypeStruct((C // 2, N), jnp.int32)

    tc1 = pl.pallas_call(
        _tc1_body, grid=(g,),
        in_specs=[rows, full, full, bias],
        out_specs=[colT, colT, colP],
        out_shape=[fTN, fTN, iPN],
        interpret=interpret)
    tc2 = pl.pallas_call(
        _tc2_body, grid=(g,),
        in_specs=[colT, colP, colT, full, full, full, full, bias],
        out_specs=[colT, colT, colP],
        out_shape=[fTN, fTN, iPN],
        interpret=interpret)
    tc3 = pl.pallas_call(
        _tc3_body, grid=(g,),
        in_specs=[colT, colT, colP, colT, full, full, full, bias],
        out_specs=rows,
        out_shape=jax.ShapeDtypeStruct((N, C), jnp.float32),
        interpret=interpret)
    return tc1, tc2, tc3


def _make_segmax(N, C, E):
    info = plsc.get_sparse_core_info()
    NC, NS = info.num_cores, info.num_subcores
    NW = NC * NS
    P = C // 2
    assert P % NW == 0
    RPW = P // NW
    assert N % _L == 0 and (RPW * N) % 8 == 0
    assert E % _CH == 0 and _CH % (_G * _L) == 0
    mesh = plsc.VectorSubcoreMesh(core_axis_name="c", subcore_axis_name="s")

    NCH = E // _CH
    assert NCH % 2 == 0

    @functools.partial(
        pl.kernel, mesh=mesh,
        out_type=jax.ShapeDtypeStruct((P * N,), jnp.int32),
        compiler_params=pltpu.CompilerParams(needs_layout_passes=False),
        scratch_types=(
            [pltpu.VMEM((N,), jnp.int32)] * RPW
            + [pltpu.VMEM((N,), jnp.int32)] * RPW
            + [pltpu.VMEM((_CH,), jnp.int32)] * 2
            + [pltpu.SemaphoreType.DMA] * 2))
    def segmax(bp_hbm, e_hbm, out_hbm, *scratch):
        b_v = scratch[:RPW]
        s_v = scratch[RPW:2 * RPW]
        ebuf = scratch[2 * RPW:2 * RPW + 2]
        sem = scratch[2 * RPW + 2:2 * RPW + 4]
        wid = lax.axis_index("s") * NC + lax.axis_index("c")
        fbase = wid * RPW * N
        for b in range(2):
            pltpu.async_copy(e_hbm.at[pl.ds(b * _CH, _CH)], ebuf[b], sem[b])
        for j in range(RPW):
            pltpu.sync_copy(bp_hbm.at[pl.ds(fbase + j * N, N)], b_v[j])

        ninf = jnp.full((_L,), _NEG_INF_PAIR, jnp.int32)

        def init_body(i, c):
            for j in range(RPW):
                s_v[j][pl.ds(i * _L, _L)] = ninf
            return c
        lax.fori_loop(0, N // _L, init_body, 0)

        def pairmax(cur_i, val_i):
            m = jnp.maximum(plsc.bitcast(cur_i, jnp.bfloat16),
                            plsc.bitcast(val_i, jnp.bfloat16))
            mi = plsc.bitcast(m, jnp.int32)
            return mi, mi != cur_i

        def unpack_idx(ev, off):
            v = ev[pl.ds(off, _L)]
            s_idx = v & 0xFFFF
            d_idx = lax.shift_right_logical(v, 16)
            return s_idx, d_idx

        def onepass(ev, off):
            s_idx, d_idx = unpack_idx(ev, off)
            vals = [plsc.load_gather(b_v[j], [s_idx]) for j in range(RPW)]
            cnt, last = plsc.scan_count(d_idx)
            dup = jnp.logical_or(jnp.logical_not(last), cnt > 0)
            curs = [plsc.load_gather(s_v[j], [d_idx]) for j in range(RPW)]
            chg = None
            for j in range(RPW):
                mi, ch = pairmax(curs[j], vals[j])
                plsc.store_scatter(s_v[j], [d_idx], mi, mask=ch)
                chg = ch if chg is None else jnp.logical_or(chg, ch)
            return jnp.logical_and(dup, chg)

        def fixpoint(ev, off):
            s_idx, d_idx = unpack_idx(ev, off)
            vals = [plsc.load_gather(b_v[j], [s_idx]) for j in range(RPW)]
            cnt, _ = plsc.scan_count(d_idx)
            hasdup = jnp.any(cnt > 0)

            def fix_body(_):
                chg = None
                for j in range(RPW):
                    cur = plsc.load_gather(s_v[j], [d_idx])
                    mi, ch = pairmax(cur, vals[j])
                    plsc.store_scatter(s_v[j], [d_idx], mi, mask=ch)
                    chg = ch if chg is None else jnp.logical_or(chg, ch)
                return jnp.logical_and(hasdup, jnp.any(chg))
            lax.while_loop(lambda r: r, fix_body, jnp.bool_(True))

        def make_group_body(ev):
            def group_body(gi, c):
                base = gi * (_G * _L)
                conflict = onepass(ev, base)
                for u in range(1, _G):
                    conflict = jnp.logical_or(conflict, onepass(ev, base + u * _L))

                @pl.when(jnp.any(conflict))
                def _():
                    for u in range(_G):
                        fixpoint(ev, base + u * _L)
                return c
            return group_body

        group_bodies = [make_group_body(ebuf[b]) for b in range(2)]

        def pair_body(cp, c):
            for b in range(2):
                ci = cp * 2 + b
                pltpu.make_async_copy(
                    e_hbm.at[pl.ds(0, _CH)], ebuf[b], sem[b]).wait()
                lax.fori_loop(0, _CH // (_G * _L), group_bodies[b], 0)
                nxt = ci + 2

                @pl.when(nxt < NCH)
                def _():
                    pltpu.async_copy(
                        e_hbm.at[pl.ds(nxt * _CH, _CH)], ebuf[b], sem[b])
            return c
        lax.fori_loop(0, NCH // 2, pair_body, 0)

        for j in range(RPW):
            pltpu.sync_copy(s_v[j], out_hbm.at[pl.ds(fbase + j * N, N)])

    return segmax


def kernel(x, edge_index, W0, b0, W1, b1, Wd, bd):
    Nin, C = x.shape
    E = edge_index.shape[1]
    src = edge_index[0]
    dst = edge_index[1]
    N = ((Nin + _NB - 1) // _NB) * _NB
    if N != Nin:
        x = jnp.pad(x, ((0, N - Nin), (0, 0)))
    if E % _CH:
        pad = _CH - E % _CH
        src = jnp.concatenate([src, jnp.broadcast_to(src[-1:], (pad,))])
        dst = jnp.concatenate([dst, jnp.broadcast_to(dst[-1:], (pad,))])
        E += pad

    U0t = (W0[:C] - W0[C:]).T
    V0t = W0[C:].T
    U1 = W1[:2 * C] - W1[2 * C:]
    V1 = W1[2 * C:]
    U1xt, U1ht = U1[:C].T, U1[C:].T
    V1xt, V1ht = V1[:C].T, V1[C:].T
    Wxt, Wh0t, Wh1t = Wd[:C].T, Wd[C:2 * C].T, Wd[2 * C:].T
    b0c = b0.reshape(C, 1)
    b1c = b1.reshape(C, 1)
    bdc = bd.reshape(C, 1)

    tc1, tc2, tc3 = _make_tc_calls(N, C)
    segmax = _make_segmax(N, C, E)

    er = E // 128
    pack = pl.pallas_call(
        _pack_edges_body,
        out_shape=jax.ShapeDtypeStruct((er, 128), jnp.int32))
    ep = pack(src.reshape(er, 128), dst.reshape(er, 128)).reshape(-1)

    xT, A0T, B0P = tc1(x, U0t, V0t, b0c)
    S0P = segmax(B0P.reshape(-1), ep).reshape(C // 2, N)
    h0T, A1T, B1P = tc2(xT, S0P, A0T, U1xt, U1ht, V1xt, V1ht, b1c)
    S1P = segmax(B1P.reshape(-1), ep).reshape(C // 2, N)
    out = tc3(xT, h0T, S1P, A1T, Wxt, Wh0t, Wh1t, bdc)
    return out[:Nin]


# speedup vs baseline: 1.9210x; 1.5062x over previous
import functools

import jax
import jax.numpy as jnp
from jax import lax
from jax.experimental import pallas as pl
from jax.experimental.pallas import tpu as pltpu
from jax.experimental.pallas import tpu_sc as plsc

_L = 16
_NB = 1024
_CH = 8192
_G = 4

_NEG_INF_PAIR = -8323200


def _pack_rows(top, bot):
    t = lax.bitcast_convert_type(top.astype(jnp.bfloat16), jnp.uint16)
    b = lax.bitcast_convert_type(bot.astype(jnp.bfloat16), jnp.uint16)
    u = t.astype(jnp.uint32) | (b.astype(jnp.uint32) << 16)
    return lax.bitcast_convert_type(u, jnp.int32)


def _unpack_rows(p):
    u = lax.bitcast_convert_type(p, jnp.uint32)
    lo = lax.bitcast_convert_type((u & 0xFFFF).astype(jnp.uint16),
                                  jnp.bfloat16).astype(jnp.float32)
    hi = lax.bitcast_convert_type((u >> 16).astype(jnp.uint16),
                                  jnp.bfloat16).astype(jnp.float32)
    return jnp.concatenate([lo, hi], axis=0)


def _pack_edges_body(src_ref, dst_ref, out_ref):
    out_ref[...] = src_ref[...] | (dst_ref[...] << 16)


def _tc1_body(x_ref, u_ref, v_ref, b_ref, xT_ref, a_ref, bp_ref):
    xT = x_ref[...].T
    xT_ref[...] = xT
    C = xT.shape[0]
    a_ref[...] = jnp.dot(u_ref[...], xT, preferred_element_type=jnp.float32) + b_ref[...]
    bm = jnp.dot(v_ref[...], xT, preferred_element_type=jnp.float32)
    bp_ref[...] = _pack_rows(bm[:C // 2], bm[C // 2:])


def _lrelu_gate(sp, a):
    P = sp.shape[0] // 2
    s = jnp.maximum(_unpack_rows(sp[:P]), _unpack_rows(sp[P:]))
    z = a + s
    h = jnp.where(z >= 0, z, 0.2 * z)
    return jnp.where(s == -jnp.inf, 0.0, h)


def _tc2_body(xT_ref, sp_ref, a0_ref, u1x_ref, u1h_ref, v1x_ref, v1h_ref,
              b_ref, h0_ref, a1_ref, b1p_ref):
    h0 = _lrelu_gate(sp_ref[...], a0_ref[...])
    h0_ref[...] = h0
    xT = xT_ref[...]
    C = xT.shape[0]
    dot = lambda w, m: jnp.dot(w, m, preferred_element_type=jnp.float32)
    a1_ref[...] = dot(u1x_ref[...], xT) + dot(u1h_ref[...], h0) + b_ref[...]
    b1 = dot(v1x_ref[...], xT) + dot(v1h_ref[...], h0)
    b1p_ref[...] = _pack_rows(b1[:C // 2], b1[C // 2:])


def _tc3_body(xT_ref, h0_ref, sp_ref, a1_ref, wx_ref, wh0_ref, wh1_ref,
              b_ref, out_ref):
    h1 = _lrelu_gate(sp_ref[...], a1_ref[...])
    xT = xT_ref[...]
    dot = lambda w, m: jnp.dot(w, m, preferred_element_type=jnp.float32)
    resT = (dot(wx_ref[...], xT) + dot(wh0_ref[...], h0_ref[...])
            + dot(wh1_ref[...], h1) + b_ref[...] + xT)
    out_ref[...] = resT.T


def _make_tc_calls(N, C, interpret=False):
    g = N // _NB
    full = pl.BlockSpec((C, C), lambda i: (0, 0))
    bias = pl.BlockSpec((C, 1), lambda i: (0, 0))
    colT = pl.BlockSpec((C, _NB), lambda i: (0, i))
    colP = pl.BlockSpec((C // 2, _NB), lambda i: (0, i))
    colS = pl.BlockSpec((C, _NB), lambda i: (0, i))
    rows = pl.BlockSpec((_NB, C), lambda i: (i, 0))
    fTN = jax.ShapeDtypeStruct((C, N), jnp.float32)
    iPN = jax.ShapeDtypeStruct((C // 2, N), jnp.int32)

    tc1 = pl.pallas_call(
        _tc1_body, grid=(g,),
        in_specs=[rows, full, full, bias],
        out_specs=[colT, colT, colP],
        out_shape=[fTN, fTN, iPN],
        interpret=interpret)
    tc2 = pl.pallas_call(
        _tc2_body, grid=(g,),
        in_specs=[colT, colS, colT, full, full, full, full, bias],
        out_specs=[colT, colT, colP],
        out_shape=[fTN, fTN, iPN],
        interpret=interpret)
    tc3 = pl.pallas_call(
        _tc3_body, grid=(g,),
        in_specs=[colT, colT, colS, colT, full, full, full, bias],
        out_specs=rows,
        out_shape=jax.ShapeDtypeStruct((N, C), jnp.float32),
        interpret=interpret)
    return tc1, tc2, tc3


def _make_segmax(N, C, E):
    info = plsc.get_sparse_core_info()
    NC, NS = info.num_cores, info.num_subcores
    NW = NC * NS
    P = C // 2
    ES = 2
    FG = NW // ES
    assert P % FG == 0
    RPW = P // FG
    assert N % _L == 0 and (RPW * N) % 8 == 0
    assert E % (2 * ES * _CH) == 0 and _CH % (_G * _L) == 0
    mesh = plsc.VectorSubcoreMesh(core_axis_name="c", subcore_axis_name="s")

    NCH = E // _CH // ES
    assert NCH % 2 == 0

    @functools.partial(
        pl.kernel, mesh=mesh,
        out_type=jax.ShapeDtypeStruct((ES * P * N,), jnp.int32),
        compiler_params=pltpu.CompilerParams(needs_layout_passes=False),
        scratch_types=(
            [pltpu.VMEM((N,), jnp.int32)] * RPW
            + [pltpu.VMEM((N,), jnp.int32)] * RPW
            + [pltpu.VMEM((_CH,), jnp.int32)] * 2
            + [pltpu.SemaphoreType.DMA] * 2))
    def segmax(bp_hbm, e_hbm, out_hbm, *scratch):
        b_v = scratch[:RPW]
        s_v = scratch[RPW:2 * RPW]
        ebuf = scratch[2 * RPW:2 * RPW + 2]
        sem = scratch[2 * RPW + 2:2 * RPW + 4]
        wid = lax.axis_index("s") * NC + lax.axis_index("c")
        fgid = wid // ES
        egid = wid % ES
        fbase = fgid * RPW * N
        cbase = egid * NCH
        for b in range(2):
            pltpu.async_copy(
                e_hbm.at[pl.ds((cbase + b) * _CH, _CH)], ebuf[b], sem[b])
        for j in range(RPW):
            pltpu.sync_copy(bp_hbm.at[pl.ds(fbase + j * N, N)], b_v[j])

        ninf = jnp.full((_L,), _NEG_INF_PAIR, jnp.int32)

        def init_body(i, c):
            for j in range(RPW):
                s_v[j][pl.ds(i * _L, _L)] = ninf
            return c
        lax.fori_loop(0, N // _L, init_body, 0)

        def pairmax(cur_i, val_i):
            m = jnp.maximum(plsc.bitcast(cur_i, jnp.bfloat16),
                            plsc.bitcast(val_i, jnp.bfloat16))
            mi = plsc.bitcast(m, jnp.int32)
            return mi, mi != cur_i

        def unpack_idx(ev, off):
            v = ev[pl.ds(off, _L)]
            s_idx = v & 0xFFFF
            d_idx = lax.shift_right_logical(v, 16)
            return s_idx, d_idx

        def onepass(ev, off):
            s_idx, d_idx = unpack_idx(ev, off)
            vals = [plsc.load_gather(b_v[j], [s_idx]) for j in range(RPW)]
            cnt, last = plsc.scan_count(d_idx)
            dup = jnp.logical_or(jnp.logical_not(last), cnt > 0)
            curs = [plsc.load_gather(s_v[j], [d_idx]) for j in range(RPW)]
            chg = None
            for j in range(RPW):
                mi, ch = pairmax(curs[j], vals[j])
                plsc.store_scatter(s_v[j], [d_idx], mi, mask=ch)
                chg = ch if chg is None else jnp.logical_or(chg, ch)
            return jnp.logical_and(dup, chg)

        def fixpoint(ev, off):
            s_idx, d_idx = unpack_idx(ev, off)
            vals = [plsc.load_gather(b_v[j], [s_idx]) for j in range(RPW)]
            cnt, _ = plsc.scan_count(d_idx)
            hasdup = jnp.any(cnt > 0)

            def fix_body(_):
                chg = None
                for j in range(RPW):
                    cur = plsc.load_gather(s_v[j], [d_idx])
                    mi, ch = pairmax(cur, vals[j])
                    plsc.store_scatter(s_v[j], [d_idx], mi, mask=ch)
                    chg = ch if chg is None else jnp.logical_or(chg, ch)
                return jnp.logical_and(hasdup, jnp.any(chg))
            lax.while_loop(lambda r: r, fix_body, jnp.bool_(True))

        def make_group_body(ev):
            def group_body(gi, c):
                base = gi * (_G * _L)
                conflict = onepass(ev, base)
                for u in range(1, _G):
                    conflict = jnp.logical_or(conflict, onepass(ev, base + u * _L))

                @pl.when(jnp.any(conflict))
                def _():
                    for u in range(_G):
                        fixpoint(ev, base + u * _L)
                return c
            return group_body

        group_bodies = [make_group_body(ebuf[b]) for b in range(2)]

        def pair_body(cp, c):
            for b in range(2):
                ci = cp * 2 + b
                pltpu.make_async_copy(
                    e_hbm.at[pl.ds(0, _CH)], ebuf[b], sem[b]).wait()
                lax.fori_loop(0, _CH // (_G * _L), group_bodies[b], 0)
                nxt = ci + 2

                @pl.when(nxt < NCH)
                def _():
                    pltpu.async_copy(
                        e_hbm.at[pl.ds((cbase + nxt) * _CH, _CH)],
                        ebuf[b], sem[b])
            return c
        lax.fori_loop(0, NCH // 2, pair_body, 0)

        obase = egid * P * N + fbase
        for j in range(RPW):
            pltpu.sync_copy(s_v[j], out_hbm.at[pl.ds(obase + j * N, N)])

    return segmax


def kernel(x, edge_index, W0, b0, W1, b1, Wd, bd):
    Nin, C = x.shape
    E = edge_index.shape[1]
    src = edge_index[0]
    dst = edge_index[1]
    N = ((Nin + _NB - 1) // _NB) * _NB
    if N != Nin:
        x = jnp.pad(x, ((0, N - Nin), (0, 0)))
    if E % (4 * _CH):
        pad = 4 * _CH - E % (4 * _CH)
        src = jnp.concatenate([src, jnp.broadcast_to(src[-1:], (pad,))])
        dst = jnp.concatenate([dst, jnp.broadcast_to(dst[-1:], (pad,))])
        E += pad

    U0t = (W0[:C] - W0[C:]).T
    V0t = W0[C:].T
    U1 = W1[:2 * C] - W1[2 * C:]
    V1 = W1[2 * C:]
    U1xt, U1ht = U1[:C].T, U1[C:].T
    V1xt, V1ht = V1[:C].T, V1[C:].T
    Wxt, Wh0t, Wh1t = Wd[:C].T, Wd[C:2 * C].T, Wd[2 * C:].T
    b0c = b0.reshape(C, 1)
    b1c = b1.reshape(C, 1)
    bdc = bd.reshape(C, 1)

    tc1, tc2, tc3 = _make_tc_calls(N, C)
    segmax = _make_segmax(N, C, E)

    er = E // 128
    pack = pl.pallas_call(
        _pack_edges_body,
        out_shape=jax.ShapeDtypeStruct((er, 128), jnp.int32))
    ep = pack(src.reshape(er, 128), dst.reshape(er, 128)).reshape(-1)

    xT, A0T, B0P = tc1(x, U0t, V0t, b0c)
    S0P = segmax(B0P.reshape(-1), ep).reshape(C, N)
    h0T, A1T, B1P = tc2(xT, S0P, A0T, U1xt, U1ht, V1xt, V1ht, b1c)
    S1P = segmax(B1P.reshape(-1), ep).reshape(C, N)
    out = tc3(xT, h0T, S1P, A1T, Wxt, Wh0t, Wh1t, bdc)
    return out[:Nin]
